# baseline (speedup 1.0000x reference)
import jax
import jax.numpy as jnp
from jax.experimental import pallas as pl
from jax.experimental.pallas import tpu as pltpu

B, DK, SEQ_DIM, HID_DIM, N, NE_DIM = 32, 512, 4096, 512, 2048, 32
NSTEP = 8
RB = N // 4
INV_SQRT_S = 1.0 / (float(SEQ_DIM) ** 0.5)


def _bitplane(xi, j, out_dtype):
    return ((xi >> (7 - j)) & 1).astype(out_dtype)


def _unpack_bitmajor(xi):
    return jnp.concatenate(
        [_bitplane(xi, j, jnp.float32) for j in range(8)], axis=1)


def _seqnet_kernel(q_ref, qok_ref, refs_ref, mask_ref, nst_ref,
                   wres_hbm, wk_hbm, ne_ref, bo_ref, out_ref,
                   a1_s, qproj_s, ku_s, scores_s, g_s, wbuf, wsem):
    i = pl.program_id(0)

    @pl.when(i == 0)
    def _fetch_all_planes():
        for s in range(8):
            pltpu.make_async_copy(
                wres_hbm.at[:, s, :], wbuf.at[s], wsem.at[s]).start()
        for s in range(8):
            pltpu.make_async_copy(
                wk_hbm.at[:, s, :], wbuf.at[8 + s], wsem.at[8 + s]).start()

    s0, s1 = 2 * i, 2 * i + 1
    pltpu.make_async_copy(wbuf.at[s0], wbuf.at[s0], wsem.at[s0]).wait()
    pltpu.make_async_copy(wbuf.at[s1], wbuf.at[s1], wsem.at[s1]).wait()
    w0, w1 = wbuf[s0], wbuf[s1]

    @pl.when(i < 4)
    def _phase1():
        qi = q_ref[...].astype(jnp.int32)

        @pl.when(i == 0)
        def _init_a1():
            qoki = qok_ref[...].astype(jnp.int32)
            for j in range(8):
                a1_s[:, j * DK:(j + 1) * DK] = (
                    _bitplane(qi, j, jnp.float32)
                    * _bitplane(qoki, j, jnp.float32) * INV_SQRT_S)

        qp0 = ((qi >> (7 - s0)) & 1).astype(jnp.float32)
        qp1 = ((qi >> (7 - s1)) & 1).astype(jnp.float32)
        contrib = (jnp.dot(qp0, w0, preferred_element_type=jnp.float32)
                   + jnp.dot(qp1, w1, preferred_element_type=jnp.float32))

        @pl.when(i == 0)
        def _qp0():
            qproj_s[...] = contrib

        @pl.when(i > 0)
        def _qpn():
            qproj_s[...] = qproj_s[...] + contrib

        kb = _unpack_bitmajor(
            refs_ref[pl.ds(i * RB, RB), :].astype(jnp.int32))
        ku_s[pl.ds(i * RB, RB), :] = kb
        scores_s[:, pl.ds(i * RB, RB)] = jax.lax.dot_general(
            a1_s[...], kb, (((1,), (1,)), ((), ())),
            preferred_element_type=jnp.float32)

    @pl.when(i >= 4)
    def _phase2():
        v0 = jax.lax.dot_general(qproj_s[...], w0, (((1,), (1,)), ((), ())),
                                 preferred_element_type=jnp.float32)
        v1 = jax.lax.dot_general(qproj_s[...], w1, (((1,), (1,)), ((), ())),
                                 preferred_element_type=jnp.float32)
        vcat = jnp.concatenate([v0, v1], axis=1)
        gj = jax.lax.dot_general(
            vcat, ku_s[:, pl.ds((i - 4) * (2 * DK), 2 * DK)],
            (((1,), (1,)), ((), ())),
            preferred_element_type=jnp.float32)

        @pl.when(i == 4)
        def _g0():
            g_s[...] = gj

        @pl.when(i > 4)
        def _gn():
            g_s[...] = g_s[...] + gj

    @pl.when(i == NSTEP - 1)
    def _epilogue():
        s = jnp.where(mask_ref[...] > 0, scores_s[...], -1e9)
        m = jnp.max(s, axis=1, keepdims=True)
        e = jnp.exp(s - m)
        w = e / jnp.sum(e, axis=1, keepdims=True)
        nef = ne_ref[...].astype(jnp.float32)
        nstf = nst_ref[...].astype(jnp.float32)
        ne0 = jnp.sum(nef[0:1, :], keepdims=True)
        ne1 = jnp.sum(nef[1:2, :], keepdims=True)
        c = ne0 * nstf[0:1, :] + ne1 * nstf[1:2, :]
        out_ref[...] = w * g_s[...] + c + bo_ref[...]


@jax.jit
def kernel(Q, Q_ok, td_refs, td_mask, td_node_state, W_res, W_k, node_embed, b_o):
    wres_3d = W_res.reshape(DK, 8, HID_DIM)
    wk_3d = W_k.reshape(DK, 8, HID_DIM)
    nst_t = td_node_state.T.astype(jnp.bfloat16)
    ne_r = node_embed.astype(jnp.bfloat16)
    mask_f = td_mask.astype(jnp.float32)

    full = lambda shape: pl.BlockSpec(shape, lambda i: (0,) * len(shape))
    out = pl.pallas_call(
        _seqnet_kernel,
        grid=(NSTEP,),
        in_specs=[
            full((B, DK)),
            full((B, DK)),
            full((N, DK)),
            full((B, N)),
            full((2, N)),
            pl.BlockSpec(memory_space=pltpu.MemorySpace.HBM),
            pl.BlockSpec(memory_space=pltpu.MemorySpace.HBM),
            full((2, NE_DIM)),
            full((1, N)),
        ],
        out_specs=full((B, N)),
        out_shape=jax.ShapeDtypeStruct((B, N), jnp.float32),
        scratch_shapes=[
            pltpu.VMEM((B, SEQ_DIM), jnp.float32),
            pltpu.VMEM((B, HID_DIM), jnp.float32),
            pltpu.VMEM((N, SEQ_DIM), jnp.float32),
            pltpu.VMEM((B, N), jnp.float32),
            pltpu.VMEM((B, N), jnp.float32),
            pltpu.VMEM((16, DK, HID_DIM), jnp.float32),
            pltpu.SemaphoreType.DMA((16,)),
        ],
    )(Q, Q_ok, td_refs, mask_f, nst_t, wres_3d, wk_3d, ne_r, b_o)
    return out


# speedup vs baseline: 1.1246x; 1.1246x over previous
import jax
import jax.numpy as jnp
from jax.experimental import pallas as pl
from jax.experimental.pallas import tpu as pltpu

B, DK, SEQ_DIM, HID_DIM, N, NE_DIM = 32, 512, 4096, 512, 2048, 32
NSTEP = 8
RB = N // 4
INV_SQRT_S = 1.0 / (float(SEQ_DIM) ** 0.5)


def _bitplane(xi, j, out_dtype):
    return ((xi >> (7 - j)) & 1).astype(out_dtype)


def _unpack_bitmajor(xi):
    return jnp.concatenate(
        [_bitplane(xi, j, jnp.float32) for j in range(8)], axis=1)


def _seqnet_kernel(q_ref, qok_ref, refs_ref, mask_ref, nst_ref,
                   wres_hbm, wk_hbm, ne_ref, bo_ref, out_ref,
                   a1_s, qproj_s, ku_s, scores_s, g_s, wbuf, wsem):
    i = pl.program_id(0)

    @pl.when(i == 0)
    def _fetch_all_planes():
        for s in range(8):
            pltpu.make_async_copy(
                wres_hbm.at[:, s, :], wbuf.at[s], wsem.at[s]).start()
        for s in range(8):
            pltpu.make_async_copy(
                wk_hbm.at[:, s, :], wbuf.at[8 + s], wsem.at[8 + s]).start()

    s0, s1 = 2 * i, 2 * i + 1
    pltpu.make_async_copy(wbuf.at[s0], wbuf.at[s0], wsem.at[s0]).wait()
    pltpu.make_async_copy(wbuf.at[s1], wbuf.at[s1], wsem.at[s1]).wait()
    w0, w1 = wbuf[s0], wbuf[s1]

    @pl.when(i < 4)
    def _phase1():
        qi = q_ref[...].astype(jnp.int32)

        @pl.when(i == 0)
        def _init_a1():
            qoki = qok_ref[...].astype(jnp.int32)
            for j in range(8):
                a1_s[:, j * DK:(j + 1) * DK] = (
                    _bitplane(qi, j, jnp.float32)
                    * _bitplane(qoki, j, jnp.float32) * INV_SQRT_S)

        qp0 = ((qi >> (7 - s0)) & 1).astype(jnp.float32)
        qp1 = ((qi >> (7 - s1)) & 1).astype(jnp.float32)
        contrib = (jnp.dot(qp0, w0, preferred_element_type=jnp.float32)
                   + jnp.dot(qp1, w1, preferred_element_type=jnp.float32))

        @pl.when(i == 0)
        def _qp0():
            qproj_s[...] = contrib

        @pl.when(i > 0)
        def _qpn():
            qproj_s[...] = qproj_s[...] + contrib

        kb = _unpack_bitmajor(
            refs_ref[pl.ds(i * RB, RB), :].astype(jnp.int32))
        ku_s[pl.ds(i * RB, RB), :] = kb
        scores_s[:, pl.ds(i * RB, RB)] = jax.lax.dot_general(
            a1_s[...], kb, (((1,), (1,)), ((), ())),
            preferred_element_type=jnp.float32)

    @pl.when(i >= 4)
    def _phase2():
        v0 = jax.lax.dot_general(qproj_s[...], w0, (((1,), (1,)), ((), ())),
                                 preferred_element_type=jnp.float32)
        v1 = jax.lax.dot_general(qproj_s[...], w1, (((1,), (1,)), ((), ())),
                                 preferred_element_type=jnp.float32)
        vcat = jnp.concatenate([v0, v1], axis=1)
        gj = jax.lax.dot_general(
            vcat, ku_s[:, pl.ds((i - 4) * (2 * DK), 2 * DK)],
            (((1,), (1,)), ((), ())),
            preferred_element_type=jnp.float32)

        @pl.when(i == 4)
        def _g0():
            g_s[...] = gj

        @pl.when(i > 4)
        def _gn():
            g_s[...] = g_s[...] + gj

    @pl.when(i == NSTEP - 1)
    def _epilogue():
        s = jnp.where(mask_ref[...] > 0, scores_s[...], -1e9)
        m = jnp.max(s, axis=1, keepdims=True)
        e = jnp.exp(s - m)
        w = e / jnp.sum(e, axis=1, keepdims=True)
        nef = ne_ref[...].astype(jnp.bfloat16).astype(jnp.float32)
        nstf = nst_ref[...].astype(jnp.bfloat16).astype(jnp.float32)
        ne0 = jnp.sum(nef[0:1, :], keepdims=True)
        ne1 = jnp.sum(nef[1:2, :], keepdims=True)
        c = ne0 * nstf[0:1, :] + ne1 * nstf[1:2, :]
        out_ref[...] = w * g_s[...] + c + bo_ref[...]


@jax.jit
def kernel(Q, Q_ok, td_refs, td_mask, td_node_state, W_res, W_k, node_embed, b_o):
    wres_3d = W_res.reshape(DK, 8, HID_DIM)
    wk_3d = W_k.reshape(DK, 8, HID_DIM)
    nst_t = td_node_state.T
    ne_r = node_embed
    mask_f = td_mask.astype(jnp.float32)

    full = lambda shape: pl.BlockSpec(shape, lambda i: (0,) * len(shape))
    out = pl.pallas_call(
        _seqnet_kernel,
        grid=(NSTEP,),
        in_specs=[
            full((B, DK)),
            full((B, DK)),
            full((N, DK)),
            full((B, N)),
            full((2, N)),
            pl.BlockSpec(memory_space=pltpu.MemorySpace.HBM),
            pl.BlockSpec(memory_space=pltpu.MemorySpace.HBM),
            full((2, NE_DIM)),
            full((1, N)),
        ],
        out_specs=full((B, N)),
        out_shape=jax.ShapeDtypeStruct((B, N), jnp.float32),
        scratch_shapes=[
            pltpu.VMEM((B, SEQ_DIM), jnp.float32),
            pltpu.VMEM((B, HID_DIM), jnp.float32),
            pltpu.VMEM((N, SEQ_DIM), jnp.float32),
            pltpu.VMEM((B, N), jnp.float32),
            pltpu.VMEM((B, N), jnp.float32),
            pltpu.VMEM((16, DK, HID_DIM), jnp.float32),
            pltpu.SemaphoreType.DMA((16,)),
        ],
    )(Q, Q_ok, td_refs, mask_f, nst_t, wres_3d, wk_3d, ne_r, b_o)
    return out
